# initial kernel scaffold (unmeasured)
import jax
import jax.numpy as jnp
from jax import lax
from jax.experimental import pallas as pl
from jax.experimental.pallas import tpu as pltpu


def kernel(
    x,
):
    def body(*refs):
        pass

    out_shape = jax.ShapeDtypeStruct(..., jnp.float32)
    return pl.pallas_call(body, out_shape=out_shape)(...)


# baseline (device time: 2190352 ns/iter reference)
import jax
import jax.numpy as jnp
from jax import lax
from jax.experimental import pallas as pl
from jax.experimental.pallas import tpu as pltpu


def kernel(x):
    m_per, n = x.shape
    half = m_per // 2
    x = x.astype(jnp.bfloat16)

    def body(x_ref, out_ref, send_sem1, recv_sem1, send_sem2, recv_sem2,
             copy_sem):
        my_x = lax.axis_index("x")
        my_y = lax.axis_index("y")
        other_x = 1 - my_x
        other_y = 1 - my_y

        barrier = pltpu.get_barrier_semaphore()
        pl.semaphore_signal(barrier, inc=1, device_id=(my_x, other_y),
                            device_id_type=pl.DeviceIdType.MESH)
        pl.semaphore_signal(barrier, inc=1, device_id=(other_x, my_y),
                            device_id_type=pl.DeviceIdType.MESH)
        pl.semaphore_wait(barrier, 2)

        local = pltpu.make_async_copy(
            x_ref, out_ref.at[pl.ds(my_y * m_per, m_per)], copy_sem)
        local.start()

        q_own = my_y * m_per + my_x * half
        p1 = pltpu.make_async_remote_copy(
            src_ref=x_ref.at[pl.ds(my_x * half, half)],
            dst_ref=out_ref.at[pl.ds(q_own, half)],
            send_sem=send_sem1,
            recv_sem=recv_sem1,
            device_id=(my_x, other_y),
            device_id_type=pl.DeviceIdType.MESH,
        )
        p1.start()
        p1.wait()

        q_fwd = other_y * m_per + my_x * half
        p2 = pltpu.make_async_remote_copy(
            src_ref=out_ref.at[pl.ds(q_fwd, half)],
            dst_ref=out_ref.at[pl.ds(q_fwd, half)],
            send_sem=send_sem2,
            recv_sem=recv_sem2,
            device_id=(other_x, my_y),
            device_id_type=pl.DeviceIdType.MESH,
        )
        p2.start()
        p2.wait()
        local.wait()

    out_shape = jax.ShapeDtypeStruct((2 * m_per, n), jnp.bfloat16)
    return pl.pallas_call(
        body,
        out_shape=out_shape,
        in_specs=[pl.BlockSpec(memory_space=pl.ANY)],
        out_specs=pl.BlockSpec(memory_space=pl.ANY),
        scratch_shapes=[
            pltpu.SemaphoreType.DMA,
            pltpu.SemaphoreType.DMA,
            pltpu.SemaphoreType.DMA,
            pltpu.SemaphoreType.DMA,
            pltpu.SemaphoreType.DMA,
        ],
        compiler_params=pltpu.CompilerParams(collective_id=0),
    )(x)


# device time: 515701 ns/iter; 4.2473x vs baseline; 4.2473x over previous
import jax
import jax.numpy as jnp
from jax import lax
from jax.experimental import pallas as pl
from jax.experimental.pallas import tpu as pltpu

CK = 1024


def kernel(x):
    m_per, n = x.shape
    half = m_per // 2
    nck = m_per // CK
    nq = half // CK

    def body(x_ref, out_ref, vin, vout, insems, outsems,
             p1send, p1recv, p2send, p2recv):
        my_x = lax.axis_index("x")
        my_y = lax.axis_index("y")
        other_x = 1 - my_x
        other_y = 1 - my_y

        barrier = pltpu.get_barrier_semaphore()
        pl.semaphore_signal(barrier, inc=1, device_id=(my_x, other_y),
                            device_id_type=pl.DeviceIdType.MESH)
        pl.semaphore_signal(barrier, inc=1, device_id=(other_x, my_y),
                            device_id_type=pl.DeviceIdType.MESH)
        pl.semaphore_wait(barrier, 2)

        q_own = my_y * m_per + my_x * half
        q_fwd = other_y * m_per + my_x * half

        src_rows = [my_x * half + c * CK for c in range(nq)]
        src_rows += [other_x * half + c * CK for c in range(nq)]

        def in_copy(c, s):
            return pltpu.make_async_copy(
                x_ref.at[pl.ds(src_rows[c], CK)], vin.at[s], insems.at[s])

        in_ops = [None] * nck
        out_ops = [None] * nck
        out_waited = [False] * nck
        p1_ops = [None] * nq

        in_ops[0] = in_copy(0, 0)
        in_ops[0].start()
        in_ops[1] = in_copy(1, 1)
        in_ops[1].start()

        for c in range(nck):
            s = c % 2
            in_ops[c].wait()
            if c >= 2 and not out_waited[c - 2]:
                out_ops[c - 2].wait()
                out_waited[c - 2] = True
            vout[s] = vin[s].astype(jnp.bfloat16)
            if c + 2 < nck:
                in_ops[c + 2] = in_copy(c + 2, s)
                in_ops[c + 2].start()
            out_ops[c] = pltpu.make_async_copy(
                vout.at[s],
                out_ref.at[pl.ds(my_y * m_per + src_rows[c], CK)],
                outsems.at[s])
            out_ops[c].start()
            if c < nq:
                out_ops[c].wait()
                out_waited[c] = True
                p1_ops[c] = pltpu.make_async_remote_copy(
                    src_ref=out_ref.at[pl.ds(q_own + c * CK, CK)],
                    dst_ref=out_ref.at[pl.ds(q_own + c * CK, CK)],
                    send_sem=p1send.at[c],
                    recv_sem=p1recv.at[c],
                    device_id=(my_x, other_y),
                    device_id_type=pl.DeviceIdType.MESH,
                )
                p1_ops[c].start()

        p2_ops = [None] * nq
        for c in range(nq):
            p1_ops[c].wait_recv()
            p2_ops[c] = pltpu.make_async_remote_copy(
                src_ref=out_ref.at[pl.ds(q_fwd + c * CK, CK)],
                dst_ref=out_ref.at[pl.ds(q_fwd + c * CK, CK)],
                send_sem=p2send.at[c],
                recv_sem=p2recv.at[c],
                device_id=(other_x, my_y),
                device_id_type=pl.DeviceIdType.MESH,
            )
            p2_ops[c].start()

        for c in range(nck):
            if not out_waited[c]:
                out_ops[c].wait()
        for c in range(nq):
            p1_ops[c].wait_send()
            p2_ops[c].wait()

    out_shape = jax.ShapeDtypeStruct((2 * m_per, n), jnp.bfloat16)
    return pl.pallas_call(
        body,
        out_shape=out_shape,
        in_specs=[pl.BlockSpec(memory_space=pl.ANY)],
        out_specs=pl.BlockSpec(memory_space=pl.ANY),
        scratch_shapes=[
            pltpu.VMEM((2, CK, n), jnp.float32),
            pltpu.VMEM((2, CK, n), jnp.bfloat16),
            pltpu.SemaphoreType.DMA((2,)),
            pltpu.SemaphoreType.DMA((2,)),
            pltpu.SemaphoreType.DMA((nq,)),
            pltpu.SemaphoreType.DMA((nq,)),
            pltpu.SemaphoreType.DMA((nq,)),
            pltpu.SemaphoreType.DMA((nq,)),
        ],
        compiler_params=pltpu.CompilerParams(collective_id=0),
    )(x)


# device time: 485272 ns/iter; 4.5137x vs baseline; 1.0627x over previous
import jax
import jax.numpy as jnp
from jax import lax
from jax.experimental import pallas as pl
from jax.experimental.pallas import tpu as pltpu

CK = 1024


def kernel(x):
    m_per, n = x.shape
    half = m_per // 2
    nck = m_per // CK
    nq = half // CK

    def body(x_ref, out_ref, vin, vout, insems, outsems,
             p1send, p1recv, p2send, p2recv):
        my_x = lax.axis_index("x")
        my_y = lax.axis_index("y")
        other_x = 1 - my_x
        other_y = 1 - my_y

        barrier = pltpu.get_barrier_semaphore()
        pl.semaphore_signal(barrier, inc=1, device_id=(my_x, other_y),
                            device_id_type=pl.DeviceIdType.MESH)
        pl.semaphore_signal(barrier, inc=1, device_id=(other_x, my_y),
                            device_id_type=pl.DeviceIdType.MESH)
        pl.semaphore_wait(barrier, 2)

        q_own = my_y * m_per + my_x * half
        q_fwd = other_y * m_per + my_x * half

        src_rows = [my_x * half + c * CK for c in range(nq)]
        src_rows += [other_x * half + c * CK for c in range(nq)]

        def in_copy(c, s):
            return pltpu.make_async_copy(
                x_ref.at[pl.ds(src_rows[c], CK)], vin.at[s], insems.at[s])

        in_ops = [None] * nck
        out_ops = [None] * nck
        out_waited = [False] * nck
        p1_ops = [None] * nq
        p2_ops = [None] * nq

        in_ops[0] = in_copy(0, 0)
        in_ops[0].start()
        in_ops[1] = in_copy(1, 1)
        in_ops[1].start()

        for c in range(nck):
            s = c % 2
            in_ops[c].wait()
            if c >= 2 and not out_waited[c - 2]:
                out_ops[c - 2].wait()
                out_waited[c - 2] = True
            vout[s] = vin[s].astype(jnp.bfloat16)
            if c + 2 < nck:
                in_ops[c + 2] = in_copy(c + 2, s)
                in_ops[c + 2].start()
            out_ops[c] = pltpu.make_async_copy(
                vout.at[s],
                out_ref.at[pl.ds(my_y * m_per + src_rows[c], CK)],
                outsems.at[s])
            out_ops[c].start()
            if c < nq:
                out_ops[c].wait()
                out_waited[c] = True
                p1_ops[c] = pltpu.make_async_remote_copy(
                    src_ref=out_ref.at[pl.ds(q_own + c * CK, CK)],
                    dst_ref=out_ref.at[pl.ds(q_own + c * CK, CK)],
                    send_sem=p1send.at[c],
                    recv_sem=p1recv.at[c],
                    device_id=(my_x, other_y),
                    device_id_type=pl.DeviceIdType.MESH,
                )
                p1_ops[c].start()

            if c >= nq:
                j = c - nq
                p1_ops[j].wait_recv()
                p2_ops[j] = pltpu.make_async_remote_copy(
                    src_ref=out_ref.at[pl.ds(q_fwd + j * CK, CK)],
                    dst_ref=out_ref.at[pl.ds(q_fwd + j * CK, CK)],
                    send_sem=p2send.at[j],
                    recv_sem=p2recv.at[j],
                    device_id=(other_x, my_y),
                    device_id_type=pl.DeviceIdType.MESH,
                )
                p2_ops[j].start()

        for c in range(nck):
            if not out_waited[c]:
                out_ops[c].wait()
        for c in range(nq):
            p1_ops[c].wait_send()
            p2_ops[c].wait()

    out_shape = jax.ShapeDtypeStruct((2 * m_per, n), jnp.bfloat16)
    return pl.pallas_call(
        body,
        out_shape=out_shape,
        in_specs=[pl.BlockSpec(memory_space=pl.ANY)],
        out_specs=pl.BlockSpec(memory_space=pl.ANY),
        scratch_shapes=[
            pltpu.VMEM((2, CK, n), jnp.float32),
            pltpu.VMEM((2, CK, n), jnp.bfloat16),
            pltpu.SemaphoreType.DMA((2,)),
            pltpu.SemaphoreType.DMA((2,)),
            pltpu.SemaphoreType.DMA((nq,)),
            pltpu.SemaphoreType.DMA((nq,)),
            pltpu.SemaphoreType.DMA((nq,)),
            pltpu.SemaphoreType.DMA((nq,)),
        ],
        compiler_params=pltpu.CompilerParams(collective_id=0),
    )(x)
